# baseline (speedup 1.0000x reference)
import functools

import jax
import jax.numpy as jnp
from jax.experimental import pallas as pl


def _chamfer_batch_kernel(n_i, bi, a_ref, b_ref, min_a_ref, min_b_ref):
    bm = b_ref[0]
    m = bm.shape[0]
    bh = bm.astype(jnp.bfloat16)
    bhf = bh.astype(jnp.float32)
    b2 = jnp.sum(bhf * bhf, axis=1, keepdims=True)
    b2_hi = b2.astype(jnp.bfloat16)
    b2_lo = (b2 - b2_hi.astype(jnp.float32)).astype(jnp.bfloat16)
    bs = (-2.0 * bhf).astype(jnp.bfloat16)
    ones_b = jnp.ones((m, 2), jnp.bfloat16)
    bm_aug = jnp.concatenate([bs, ones_b, b2_hi, b2_lo], axis=1)

    def step(i, colmin):
        a = a_ref[0, pl.ds(i * bi, bi), :]
        ah = a.astype(jnp.bfloat16)
        ahf = ah.astype(jnp.float32)
        a2 = jnp.sum(ahf * ahf, axis=1, keepdims=True)
        a2_hi = a2.astype(jnp.bfloat16)
        a2_lo = (a2 - a2_hi.astype(jnp.float32)).astype(jnp.bfloat16)
        ones_a = jnp.ones((bi, 2), jnp.bfloat16)
        a_aug = jnp.concatenate([ah, a2_hi, a2_lo, ones_a], axis=1)
        d2 = jax.lax.dot_general(
            a_aug, bm_aug, (((1,), (1,)), ((), ())),
            preferred_element_type=jnp.float32,
        )
        rowmin = jnp.min(d2, axis=1, keepdims=True)
        min_a_ref[0, pl.ds(i * bi, bi), :] = jnp.sqrt(jnp.maximum(rowmin, 0.0))
        return jnp.minimum(colmin, jnp.min(d2, axis=0))

    init = jnp.full((m,), jnp.inf, jnp.float32)
    colmin = jax.lax.fori_loop(0, n_i, step, init)
    min_b_ref[0, 0, :] = jnp.sqrt(jnp.maximum(colmin, 0.0))


def kernel(A, B):
    bt, n, d = A.shape
    m = B.shape[1]
    bi = 512
    n_i = n // bi

    min_a, min_b = pl.pallas_call(
        functools.partial(_chamfer_batch_kernel, n_i, bi),
        grid=(bt,),
        in_specs=[
            pl.BlockSpec((1, n, d), lambda b: (b, 0, 0)),
            pl.BlockSpec((1, m, d), lambda b: (b, 0, 0)),
        ],
        out_specs=[
            pl.BlockSpec((1, n, 1), lambda b: (b, 0, 0)),
            pl.BlockSpec((1, 1, m), lambda b: (b, 0, 0)),
        ],
        out_shape=[
            jax.ShapeDtypeStruct((bt, n, 1), jnp.float32),
            jax.ShapeDtypeStruct((bt, 1, m), jnp.float32),
        ],
    )(A, B)
    min_a = min_a.reshape(bt, n)
    min_b = min_b.reshape(bt, m)
    chamfer = jnp.mean(min_a, axis=1) + jnp.mean(min_b, axis=1)
    return jnp.mean(chamfer) / 12.8


# speedup vs baseline: 1.1815x; 1.1815x over previous
import functools

import jax
import jax.numpy as jnp
from jax.experimental import pallas as pl


def _chamfer_batch_kernel(n_i, bi, a_ref, b_ref, min_a_ref, min_b_ref):
    bm = b_ref[0]
    m = bm.shape[0]
    bh = bm.astype(jnp.bfloat16)
    bhf = bh.astype(jnp.float32)
    b2 = jnp.sum(bhf * bhf, axis=1, keepdims=True)
    b2_hi = b2.astype(jnp.bfloat16)
    b2_lo = (b2 - b2_hi.astype(jnp.float32)).astype(jnp.bfloat16)
    bs = (-2.0 * bhf).astype(jnp.bfloat16)
    ones_b = jnp.ones((m, 2), jnp.bfloat16)
    bm_aug = jnp.concatenate([bs, ones_b, b2_hi, b2_lo], axis=1)

    def step(i, colmin):
        a = a_ref[0, i * bi:(i + 1) * bi, :]
        ah = a.astype(jnp.bfloat16)
        ahf = ah.astype(jnp.float32)
        a2 = jnp.sum(ahf * ahf, axis=1, keepdims=True)
        a2_hi = a2.astype(jnp.bfloat16)
        a2_lo = (a2 - a2_hi.astype(jnp.float32)).astype(jnp.bfloat16)
        ones_a = jnp.ones((bi, 2), jnp.bfloat16)
        a_aug = jnp.concatenate([ah, a2_hi, a2_lo, ones_a], axis=1)
        d2 = jax.lax.dot_general(
            a_aug, bm_aug, (((1,), (1,)), ((), ())),
            preferred_element_type=jnp.float32,
        )
        rowmin = jnp.min(d2, axis=1, keepdims=True)
        min_a_ref[0, i * bi:(i + 1) * bi, :] = jnp.sqrt(jnp.maximum(rowmin, 0.0))
        return jnp.minimum(colmin, jnp.min(d2, axis=0)) if colmin is not None \
            else jnp.min(d2, axis=0)

    colmin = None
    for i in range(n_i):
        colmin = step(i, colmin)
    min_b_ref[0, 0, :] = jnp.sqrt(jnp.maximum(colmin, 0.0))


def kernel(A, B):
    bt, n, d = A.shape
    m = B.shape[1]
    bi = 512
    n_i = n // bi

    min_a, min_b = pl.pallas_call(
        functools.partial(_chamfer_batch_kernel, n_i, bi),
        grid=(bt,),
        in_specs=[
            pl.BlockSpec((1, n, d), lambda b: (b, 0, 0)),
            pl.BlockSpec((1, m, d), lambda b: (b, 0, 0)),
        ],
        out_specs=[
            pl.BlockSpec((1, n, 1), lambda b: (b, 0, 0)),
            pl.BlockSpec((1, 1, m), lambda b: (b, 0, 0)),
        ],
        out_shape=[
            jax.ShapeDtypeStruct((bt, n, 1), jnp.float32),
            jax.ShapeDtypeStruct((bt, 1, m), jnp.float32),
        ],
    )(A, B)
    min_a = min_a.reshape(bt, n)
    min_b = min_b.reshape(bt, m)
    chamfer = jnp.mean(min_a, axis=1) + jnp.mean(min_b, axis=1)
    return jnp.mean(chamfer) / 12.8
